# baseline (speedup 1.0000x reference)
import functools
import math

import jax
import jax.numpy as jnp
import numpy as np
from jax import lax
from jax.experimental import pallas as pl
from jax.experimental.pallas import tpu as pltpu
from jax.experimental.pallas import tpu_sc as plsc

H = 128
N = 10000
E = 320000
K = 8000
NUM_CLASSES = 10
INV = 1.0 / math.sqrt(1.0 + 1e-5)
SQ = float(np.sqrt(np.float32(1.0 + 1e-5)))

NSUB = 16
CH = 128
SUP = 8
NSUP = 20
CHUNKS = SUP * NSUP
EP = NSUB * CH * CHUNKS
NACC = 10240
RPS = NACC // NSUB


def _dot(a, b, prec=None):
    return lax.dot_general(a, b, (((1,), (0,)), ((), ())), precision=prec)


def _tcb_body(agg_ref, x1_ref, x2_ref, w1r_ref, w1s_ref, b1_ref, g1_ref,
              bb1_ref, w2r_ref, w2s_ref, b2_ref, g2_ref, bb2_ref,
              p1_ref, p2_ref, h1_ref, h2_ref, sc_ref):
    gc1 = _dot(agg_ref[0, :N, :], w1r_ref[...]) + _dot(x1_ref[...], w1s_ref[...]) \
        + b1_ref[...][None, :]
    h1 = jax.nn.relu(gc1 * g1_ref[...][None, :] / SQ + bb1_ref[...][None, :])
    gc2 = _dot(agg_ref[1, :N, :], w2r_ref[...]) + _dot(x2_ref[...], w2s_ref[...]) \
        + b2_ref[...][None, :]
    h2 = jax.nn.relu(gc2 * g2_ref[...][None, :] / SQ + bb2_ref[...][None, :])
    h1_ref[...] = h1
    h2_ref[...] = h2
    sc_ref[...] = jnp.tanh(_dot(h1, p1_ref[...]) + _dot(h2, p2_ref[...]))


def _tcb(agg, x1, x2, w1r, w1s, b1, g1, bb1, w2r, w2s, b2, g2, bb2, p1u, p2u):
    return pl.pallas_call(
        _tcb_body,
        out_shape=(jax.ShapeDtypeStruct((N, H), jnp.float32),
                   jax.ShapeDtypeStruct((N, H), jnp.float32),
                   jax.ShapeDtypeStruct((N, 1), jnp.float32)),
    )(agg, x1, x2, w1r, w1s, b1, g1, bb1, w2r, w2s, b2, g2, bb2, p1u, p2u)


def _zmask_body(h1_ref, h2_ref, sv_ref, z1_ref, z2_ref):
    sv = sv_ref[...]
    z1_ref[...] = h1_ref[...] * sv
    z2_ref[...] = h2_ref[...] * sv


def _zmask(h1, h2, sv):
    return pl.pallas_call(
        _zmask_body,
        out_shape=(jax.ShapeDtypeStruct((N, H), jnp.float32),
                   jax.ShapeDtypeStruct((N, H), jnp.float32)),
    )(h1, h2, sv)


def _tcd_body(agg_ref, z1_ref, z2_ref, w3r_ref, w3s_ref, b3_ref, g3_ref,
              bb3_ref, w4r_ref, w4s_ref, b4_ref, g4_ref, bb4_ref, sel_ref,
              g5_ref, b5_ref, lw_ref, lb_ref, out_ref):
    sel = sel_ref[...]
    gc3 = _dot(agg_ref[0, :N, :], w3r_ref[...]) + _dot(z1_ref[...], w3s_ref[...]) \
        + b3_ref[...][None, :]
    o1 = jax.nn.relu(gc3 * g3_ref[...][None, :] / SQ + bb3_ref[...][None, :]) * sel
    gc4 = _dot(agg_ref[1, :N, :], w4r_ref[...]) + _dot(z2_ref[...], w4s_ref[...]) \
        + b4_ref[...][None, :]
    o2 = jax.nn.relu(gc4 * g4_ref[...][None, :] / SQ + bb4_ref[...][None, :]) * sel
    xg = [jnp.max(o1, axis=0, keepdims=True),
          jnp.sum(o1, axis=0, keepdims=True) * (1.0 / K),
          jnp.max(o2, axis=0, keepdims=True),
          jnp.sum(o2, axis=0, keepdims=True) * (1.0 / K)]
    logits = lb_ref[...][None, :]
    for c in range(4):
        g5c = g5_ref[c * H:(c + 1) * H]
        b5c = b5_ref[c * H:(c + 1) * H]
        xgc = (xg[c] * g5c[None, :]) / SQ + b5c[None, :]
        logits = logits + _dot(xgc, lw_ref[c * H:(c + 1) * H, :])
    m = jnp.max(logits, axis=1, keepdims=True)
    e = jnp.exp(logits - m)
    out_ref[...] = e / jnp.sum(e, axis=1, keepdims=True)


def _tcd(agg, z1, z2, w3r, w3s, b3, g3, bb3, w4r, w4s, b4, g4, bb4, sel,
         g5, b5, lw, lb):
    return pl.pallas_call(
        _tcd_body,
        out_shape=jax.ShapeDtypeStruct((1, NUM_CLASSES), jnp.float32),
    )(agg, z1, z2, w3r, w3s, b3, g3, bb3, w4r, w4s, b4, g4, bb4, sel,
      g5, b5, lw, lb)


@functools.partial(
    pl.kernel,
    out_type=jax.ShapeDtypeStruct((2, NACC, H), jnp.float32),
    mesh=plsc.VectorSubcoreMesh(core_axis_name="c", subcore_axis_name="s"),
    scratch_types=[
        pltpu.VMEM((SUP, CH), jnp.int32),
        pltpu.VMEM((SUP, CH), jnp.int32),
        pltpu.VMEM((CH, H), jnp.float32),
        pltpu.VMEM((CH, H), jnp.float32),
        pltpu.VMEM_SHARED((NACC, H), jnp.float32),
        pltpu.SemaphoreType.DMA,
        pltpu.SemaphoreType.DMA,
    ],
)
def _sc_segsum2(y1_hbm, src1_hbm, dst1_hbm, y2_hbm, src2_hbm, dst2_hbm,
                zeros_hbm, out_hbm, srcb, dstb, rowsa, rowsb, acc, sema, semb):
    c = lax.axis_index("c")
    s = lax.axis_index("s")
    rows = (rowsa, rowsb)
    sems = (sema, semb)

    def run(y_hbm, src_hbm, dst_hbm, out_m):
        rbase = s * RPS
        pltpu.sync_copy(zeros_hbm.at[pl.ds(rbase, RPS)], acc.at[pl.ds(rbase, RPS)])
        plsc.subcore_barrier()
        sbase = s * CHUNKS

        @pl.loop(0, NSUP)
        def _(t):
            soff = sbase + t * SUP
            pltpu.sync_copy(src_hbm.at[pl.ds(soff, SUP)], srcb)
            pltpu.sync_copy(dst_hbm.at[pl.ds(soff, SUP)], dstb)
            pltpu.async_copy(y_hbm.at[srcb.at[0]], rows[0], sems[0])
            for j in range(SUP):
                if j < SUP - 1:
                    pltpu.async_copy(y_hbm.at[srcb.at[j + 1]],
                                     rows[(j + 1) % 2], sems[(j + 1) % 2])
                pltpu.make_async_copy(y_hbm.at[srcb.at[j]],
                                      rows[j % 2], sems[j % 2]).wait()
                pltpu.sync_copy(rows[j % 2], acc.at[dstb.at[j]], add=True)

        plsc.subcore_barrier()
        pltpu.sync_copy(acc.at[pl.ds(rbase, RPS)], out_m.at[pl.ds(rbase, RPS)])

    @pl.when(c == 0)
    def _():
        run(y1_hbm, src1_hbm, dst1_hbm, out_hbm.at[0])

    @pl.when(c == 1)
    def _():
        run(y2_hbm, src2_hbm, dst2_hbm, out_hbm.at[1])


def _pad_edges(ei):
    src = jnp.concatenate([ei[0], jnp.zeros((EP - E,), jnp.int32)])
    dst = jnp.concatenate([ei[1], jnp.full((EP - E,), N, jnp.int32)])
    return src.reshape(EP // CH, CH), dst.reshape(EP // CH, CH)


def kernel(x1, edge_index1, x2, edge_index2, params):
    p = params
    src1, dst1 = _pad_edges(edge_index1)
    src2, dst2 = _pad_edges(edge_index2)
    zeros = jnp.zeros((NACC, H), jnp.float32)
    p1u = (p['p1'] / jnp.linalg.norm(p['p1']))[:, None]
    p2u = (p['p2'] / jnp.linalg.norm(p['p2']))[:, None]

    agg = _sc_segsum2(x1, src1, dst1, x2, src2, dst2, zeros)
    h1, h2, score = _tcb(agg, x1, x2, p['c1_Wr'], p['c1_Ws'], p['c1_b'],
                         p['n1_g'], p['n1_b'], p['c2_Wr'], p['c2_Ws'],
                         p['c2_b'], p['n2_g'], p['n2_b'], p1u, p2u)

    score1 = score[:, 0]
    topv, perm = lax.top_k(score1, K)
    sel = jnp.zeros((N,), jnp.float32).at[perm].set(1.0)
    sv = (score1 * sel)[:, None]

    z1, z2 = _zmask(h1, h2, sv)
    agg2 = _sc_segsum2(z1, src1, dst1, z2, src2, dst2, zeros)
    probs = _tcd(agg2, z1, z2, p['c3_Wr'], p['c3_Ws'], p['c3_b'], p['n3_g'],
                 p['n3_b'], p['c4_Wr'], p['c4_Ws'], p['c4_b'], p['n4_g'],
                 p['n4_b'], sel[:, None], p['n5_g'], p['n5_b'],
                 p['lin_W'], p['lin_b'])
    return probs, topv


# speedup vs baseline: 1.3896x; 1.3896x over previous
import functools
import math

import jax
import jax.numpy as jnp
import numpy as np
from jax import lax
from jax.experimental import pallas as pl
from jax.experimental.pallas import tpu as pltpu
from jax.experimental.pallas import tpu_sc as plsc

H = 128
N = 10000
E = 320000
K = 8000
NUM_CLASSES = 10
INV = 1.0 / math.sqrt(1.0 + 1e-5)
SQ = float(np.sqrt(np.float32(1.0 + 1e-5)))

NSUB = 16
CH = 128
CHUNKS = 158
EP = NSUB * CH * CHUNKS
NACC = 10240
RPS = NACC // NSUB


def _dot(a, b, prec=None):
    return lax.dot_general(a, b, (((1,), (0,)), ((), ())), precision=prec)


def _tcb_body(agg_ref, x1_ref, x2_ref, w1r_ref, w1s_ref, b1_ref, g1_ref,
              bb1_ref, w2r_ref, w2s_ref, b2_ref, g2_ref, bb2_ref,
              p1_ref, p2_ref, h1_ref, h2_ref, sc_ref):
    gc1 = _dot(agg_ref[0, :N, :], w1r_ref[...]) + _dot(x1_ref[...], w1s_ref[...]) \
        + b1_ref[...][None, :]
    h1 = jax.nn.relu(gc1 * g1_ref[...][None, :] / SQ + bb1_ref[...][None, :])
    gc2 = _dot(agg_ref[1, :N, :], w2r_ref[...]) + _dot(x2_ref[...], w2s_ref[...]) \
        + b2_ref[...][None, :]
    h2 = jax.nn.relu(gc2 * g2_ref[...][None, :] / SQ + bb2_ref[...][None, :])
    h1_ref[...] = h1
    h2_ref[...] = h2
    sc_ref[...] = jnp.tanh(_dot(h1, p1_ref[...]) + _dot(h2, p2_ref[...]))


def _tcb(agg, x1, x2, w1r, w1s, b1, g1, bb1, w2r, w2s, b2, g2, bb2, p1u, p2u):
    return pl.pallas_call(
        _tcb_body,
        out_shape=(jax.ShapeDtypeStruct((N, H), jnp.float32),
                   jax.ShapeDtypeStruct((N, H), jnp.float32),
                   jax.ShapeDtypeStruct((N, 1), jnp.float32)),
    )(agg, x1, x2, w1r, w1s, b1, g1, bb1, w2r, w2s, b2, g2, bb2, p1u, p2u)


def _zmask_body(h1_ref, h2_ref, sv_ref, z1_ref, z2_ref):
    sv = sv_ref[...]
    z1_ref[...] = h1_ref[...] * sv
    z2_ref[...] = h2_ref[...] * sv


def _zmask(h1, h2, sv):
    return pl.pallas_call(
        _zmask_body,
        out_shape=(jax.ShapeDtypeStruct((N, H), jnp.float32),
                   jax.ShapeDtypeStruct((N, H), jnp.float32)),
    )(h1, h2, sv)


def _tcd_body(agg_ref, z1_ref, z2_ref, w3r_ref, w3s_ref, b3_ref, g3_ref,
              bb3_ref, w4r_ref, w4s_ref, b4_ref, g4_ref, bb4_ref, sel_ref,
              g5_ref, b5_ref, lw_ref, lb_ref, out_ref):
    sel = sel_ref[...]
    gc3 = _dot(agg_ref[0, :N, :], w3r_ref[...]) + _dot(z1_ref[...], w3s_ref[...]) \
        + b3_ref[...][None, :]
    o1 = jax.nn.relu(gc3 * g3_ref[...][None, :] / SQ + bb3_ref[...][None, :]) * sel
    gc4 = _dot(agg_ref[1, :N, :], w4r_ref[...]) + _dot(z2_ref[...], w4s_ref[...]) \
        + b4_ref[...][None, :]
    o2 = jax.nn.relu(gc4 * g4_ref[...][None, :] / SQ + bb4_ref[...][None, :]) * sel
    xg = [jnp.max(o1, axis=0, keepdims=True),
          jnp.sum(o1, axis=0, keepdims=True) * (1.0 / K),
          jnp.max(o2, axis=0, keepdims=True),
          jnp.sum(o2, axis=0, keepdims=True) * (1.0 / K)]
    logits = lb_ref[...][None, :]
    for c in range(4):
        g5c = g5_ref[c * H:(c + 1) * H]
        b5c = b5_ref[c * H:(c + 1) * H]
        xgc = (xg[c] * g5c[None, :]) / SQ + b5c[None, :]
        logits = logits + _dot(xgc, lw_ref[c * H:(c + 1) * H, :])
    m = jnp.max(logits, axis=1, keepdims=True)
    e = jnp.exp(logits - m)
    out_ref[...] = e / jnp.sum(e, axis=1, keepdims=True)


def _tcd(agg, z1, z2, w3r, w3s, b3, g3, bb3, w4r, w4s, b4, g4, bb4, sel,
         g5, b5, lw, lb):
    return pl.pallas_call(
        _tcd_body,
        out_shape=jax.ShapeDtypeStruct((1, NUM_CLASSES), jnp.float32),
    )(agg, z1, z2, w3r, w3s, b3, g3, bb3, w4r, w4s, b4, g4, bb4, sel,
      g5, b5, lw, lb)


@functools.partial(
    pl.kernel,
    out_type=jax.ShapeDtypeStruct((2, NACC, H), jnp.float32),
    mesh=plsc.VectorSubcoreMesh(core_axis_name="c", subcore_axis_name="s"),
    scratch_types=[
        pltpu.VMEM((CH,), jnp.int32),
        pltpu.VMEM((CH,), jnp.int32),
        pltpu.VMEM((CH, H), jnp.float32),
        pltpu.VMEM((CH,), jnp.int32),
        pltpu.VMEM((CH,), jnp.int32),
        pltpu.VMEM((CH, H), jnp.float32),
        pltpu.VMEM_SHARED((NACC, H), jnp.float32),
        pltpu.SemaphoreType.DMA,
        pltpu.SemaphoreType.DMA,
    ],
)
def _sc_segsum2(y1_hbm, src1_hbm, dst1_hbm, y2_hbm, src2_hbm, dst2_hbm,
                zeros_hbm, out_hbm, srca, dsta, rowsa, srcb, dstb, rowsb,
                acc, sema, semb):
    c = lax.axis_index("c")
    s = lax.axis_index("s")

    def run(y_hbm, src_hbm, dst_hbm, out_m):
        rbase = s * RPS
        pltpu.sync_copy(zeros_hbm.at[pl.ds(rbase, RPS)], acc.at[pl.ds(rbase, RPS)])
        plsc.subcore_barrier()
        ebase = s * (CHUNKS * CH)

        pltpu.sync_copy(src_hbm.at[pl.ds(ebase, CH)], srca)
        pltpu.sync_copy(dst_hbm.at[pl.ds(ebase, CH)], dsta)
        pltpu.async_copy(y_hbm.at[srca], rowsa, sema)

        @pl.loop(0, CHUNKS // 2)
        def _(t):
            offb = ebase + (2 * t + 1) * CH
            pltpu.sync_copy(src_hbm.at[pl.ds(offb, CH)], srcb)
            pltpu.sync_copy(dst_hbm.at[pl.ds(offb, CH)], dstb)
            pltpu.async_copy(y_hbm.at[srcb], rowsb, semb)
            pltpu.make_async_copy(y_hbm.at[srca], rowsa, sema).wait()
            pltpu.sync_copy(rowsa, acc.at[dsta], add=True)

            @pl.when(t < CHUNKS // 2 - 1)
            def _():
                offa = ebase + (2 * t + 2) * CH
                pltpu.sync_copy(src_hbm.at[pl.ds(offa, CH)], srca)
                pltpu.sync_copy(dst_hbm.at[pl.ds(offa, CH)], dsta)
                pltpu.async_copy(y_hbm.at[srca], rowsa, sema)

            pltpu.make_async_copy(y_hbm.at[srcb], rowsb, semb).wait()
            pltpu.sync_copy(rowsb, acc.at[dstb], add=True)

        plsc.subcore_barrier()
        pltpu.sync_copy(acc.at[pl.ds(rbase, RPS)], out_m.at[pl.ds(rbase, RPS)])

    @pl.when(c == 0)
    def _():
        run(y1_hbm, src1_hbm, dst1_hbm, out_hbm.at[0])

    @pl.when(c == 1)
    def _():
        run(y2_hbm, src2_hbm, dst2_hbm, out_hbm.at[1])


def _pad_edges(ei):
    src = jnp.concatenate([ei[0], jnp.zeros((EP - E,), jnp.int32)])
    dst = jnp.concatenate([ei[1], jnp.full((EP - E,), N, jnp.int32)])
    return src, dst


def kernel(x1, edge_index1, x2, edge_index2, params):
    p = params
    src1, dst1 = _pad_edges(edge_index1)
    src2, dst2 = _pad_edges(edge_index2)
    zeros = jnp.zeros((NACC, H), jnp.float32)
    p1u = (p['p1'] / jnp.linalg.norm(p['p1']))[:, None]
    p2u = (p['p2'] / jnp.linalg.norm(p['p2']))[:, None]

    agg = _sc_segsum2(x1, src1, dst1, x2, src2, dst2, zeros)
    h1, h2, score = _tcb(agg, x1, x2, p['c1_Wr'], p['c1_Ws'], p['c1_b'],
                         p['n1_g'], p['n1_b'], p['c2_Wr'], p['c2_Ws'],
                         p['c2_b'], p['n2_g'], p['n2_b'], p1u, p2u)

    score1 = score[:, 0]
    topv, perm = lax.top_k(score1, K)
    sel = jnp.zeros((N,), jnp.float32).at[perm].set(1.0)
    sv = (score1 * sel)[:, None]

    z1, z2 = _zmask(h1, h2, sv)
    agg2 = _sc_segsum2(z1, src1, dst1, z2, src2, dst2, zeros)
    probs = _tcd(agg2, z1, z2, p['c3_Wr'], p['c3_Ws'], p['c3_b'], p['n3_g'],
                 p['n3_b'], p['c4_Wr'], p['c4_Ws'], p['c4_b'], p['n4_g'],
                 p['n4_b'], sel[:, None], p['n5_g'], p['n5_b'],
                 p['lin_W'], p['lin_b'])
    return probs, topv
